# baseline (speedup 1.0000x reference)
import functools

import jax
import jax.numpy as jnp
from jax import lax
from jax.experimental import pallas as pl
from jax.experimental.pallas import tpu as pltpu
from jax.experimental.pallas import tpu_sc as plsc

SRC = 4.0
D_OUT = 64
NC, NS, LANES = 2, 16, 16
NW = NC * NS
VOL = 128 * 128 * 128
N_PIX = 4 * 128 * 128
PIX_PER_W = N_PIX // NW
GROUPS = PIX_PER_W // LANES


def _set_matrix(rtvec):
    B = rtvec.shape[0]
    rx, ry, rz = rtvec[:, 0], rtvec[:, 1], rtvec[:, 2]
    tx, ty, tz = rtvec[:, 3], rtvec[:, 4], rtvec[:, 5]
    z = jnp.zeros(B, dtype=rtvec.dtype); o = jnp.ones(B, dtype=rtvec.dtype)
    cx, sx = jnp.cos(rx), jnp.sin(rx)
    cy, sy = jnp.cos(ry), jnp.sin(ry)
    cz, sz = jnp.cos(rz), jnp.sin(rz)
    Rx = jnp.stack([o, z, z, z, z, cx, -sx, z, z, sx, cx, z, z, z, z, o], axis=1).reshape(B, 4, 4)
    Ry = jnp.stack([cy, z, sy, z, z, o, z, z, -sy, z, cy, z, z, z, z, o], axis=1).reshape(B, 4, 4)
    Rz = jnp.stack([cz, -sz, z, z, sz, cz, z, z, z, z, o, z, z, z, z, o], axis=1).reshape(B, 4, 4)
    T = jnp.stack([o, z, z, tx, z, o, z, ty, z, z, o, tz, z, z, z, o], axis=1).reshape(B, 4, 4)
    rot = jnp.einsum('bij,bjk->bik', jnp.einsum('bij,bjk->bik', Rz, Ry), Rx)
    M = jnp.einsum('bij,bjk->bik', rot, T)
    return M[:, :3, :]


def _raydist_range(M, pt, src):
    pt = pt - M[:, :3, 3][:, None, :]
    invR = jnp.linalg.inv(M[:, :3, :3])
    inv_pt = jnp.einsum('bnc,bcd->bnd', pt, invR)
    inv_pt = inv_pt.at[:, :, 2].set(src - inv_pt[:, :, 2])
    flat = inv_pt.reshape(-1, 3)
    d = jnp.sqrt(flat[:, 0] ** 2 + flat[:, 1] ** 2 + flat[:, 2] ** 2)
    return jnp.min(d), jnp.max(d)


def _sc_body(vol_hbm, crd_hbm, out_hbm, crd_v, idx_buf, w_buf, val_buf, out_v, sem):
    cid = lax.axis_index("c")
    sid = lax.axis_index("s")
    wid = sid * NC + cid
    b = wid // 8
    sub = wid - b * 8
    vol_base = b * VOL

    def group(g, _):
        row = b * 1024 + (sub * 16 + g // 8) * 8 + (g - (g // 8) * 8)
        pltpu.sync_copy(crd_hbm.at[row], crd_v)

        def phase_a(d, _):
            vx = crd_v[0, d]
            vy = crd_v[1, d]
            vz = crd_v[2, d]
            x0i = jnp.clip(lax.convert_element_type(vx, jnp.int32), 0, 127)
            y0i = jnp.clip(lax.convert_element_type(vy, jnp.int32), 0, 127)
            z0i = jnp.clip(lax.convert_element_type(vz, jnp.int32), 0, 127)
            x0f = lax.convert_element_type(x0i, jnp.float32)
            y0f = lax.convert_element_type(y0i, jnp.float32)
            z0f = lax.convert_element_type(z0i, jnp.float32)
            x1f = jnp.minimum(x0f + 1.0, 127.0)
            y1f = jnp.minimum(y0f + 1.0, 127.0)
            z1f = jnp.minimum(z0f + 1.0, 127.0)
            x1i = lax.convert_element_type(x1f, jnp.int32)
            y1i = lax.convert_element_type(y1f, jnp.int32)
            z1i = lax.convert_element_type(z1f, jnp.int32)
            wx0 = x1f - vx; wx1 = vx - x0f
            wy0 = y1f - vy; wy1 = vy - y0f
            wz0 = z1f - vz; wz1 = vz - z0f
            inb = ((vx >= 0.0) & (vx <= 128.0) & (vy >= 0.0) & (vy <= 128.0)
                   & (vz >= 0.0) & (vz <= 128.0))
            msk = jnp.where(inb, 1.0, 0.0)
            wz0 = wz0 * msk
            wz1 = wz1 * msk
            r00 = vol_base + z0i * 16384 + y0i * 128
            r01 = vol_base + z0i * 16384 + y1i * 128
            r10 = vol_base + z1i * 16384 + y0i * 128
            r11 = vol_base + z1i * 16384 + y1i * 128
            idx_buf[d, 0:16] = r00 + x0i
            idx_buf[d, 16:32] = r00 + x1i
            idx_buf[d, 32:48] = r01 + x0i
            idx_buf[d, 48:64] = r01 + x1i
            idx_buf[d, 64:80] = r10 + x0i
            idx_buf[d, 80:96] = r10 + x1i
            idx_buf[d, 96:112] = r11 + x0i
            idx_buf[d, 112:128] = r11 + x1i
            w_buf[d, 0:16] = wx0
            w_buf[d, 16:32] = wx1
            w_buf[d, 32:48] = wy0
            w_buf[d, 48:64] = wy1
            w_buf[d, 64:80] = wz0
            w_buf[d, 80:96] = wz1
            return 0

        lax.fori_loop(0, D_OUT, phase_a, 0)

        def gather_serial(d, _):
            pltpu.async_copy(vol_hbm.at[idx_buf.at[d]], val_buf.at[d], sem).wait()
            return 0

        lax.fori_loop(0, D_OUT, gather_serial, 0)

        def phase_c(d, acc):
            wx0 = w_buf[d, 0:16]; wx1 = w_buf[d, 16:32]
            wy0 = w_buf[d, 32:48]; wy1 = w_buf[d, 48:64]
            wz0 = w_buf[d, 64:80]; wz1 = w_buf[d, 80:96]
            va = val_buf[d, 0:16]; vb = val_buf[d, 16:32]
            vc = val_buf[d, 32:48]; vd = val_buf[d, 48:64]
            ve = val_buf[d, 64:80]; vf = val_buf[d, 80:96]
            vg = val_buf[d, 96:112]; vh = val_buf[d, 112:128]
            bot = wy0 * (wx0 * va + wx1 * vb) + wy1 * (wx0 * vc + wx1 * vd)
            top = wy0 * (wx0 * ve + wx1 * vf) + wy1 * (wx0 * vg + wx1 * vh)
            return acc + (wz0 * bot + wz1 * top)

        acc = lax.fori_loop(0, D_OUT, phase_c, jnp.zeros((LANES,), jnp.float32))
        out_v[pl.ds(g * 16, 16)] = acc * (1.0 / D_OUT)
        return 0

    lax.fori_loop(0, GROUPS, group, 0)
    pltpu.sync_copy(out_v, out_hbm.at[pl.ds(wid * PIX_PER_W, PIX_PER_W)])


_MESH = plsc.VectorSubcoreMesh(
    core_axis_name="c", subcore_axis_name="s", num_cores=NC, num_subcores=NS)

_projector = functools.partial(
    pl.kernel,
    out_type=jax.ShapeDtypeStruct((N_PIX,), jnp.float32),
    mesh=_MESH,
    scratch_types=[
        pltpu.VMEM((3, D_OUT, LANES), jnp.float32),
        pltpu.VMEM((D_OUT, 128), jnp.int32),
        pltpu.VMEM((D_OUT, 96), jnp.float32),
        pltpu.VMEM((D_OUT, 128), jnp.float32),
        pltpu.VMEM((PIX_PER_W,), jnp.float32),
        pltpu.SemaphoreType.DMA,
    ],
)(_sc_body)


def _coords(M, scale, oH, oW, B):
    zs = jnp.linspace(-1.0, 1.0, D_OUT)
    ys = jnp.linspace(-1.0, 1.0, oH)
    xs = jnp.linspace(-1.0, 1.0, oW)
    gz, gy, gx = jnp.meshgrid(zs, ys, xs, indexing='ij')
    pts = jnp.stack([gx, gy, gz, jnp.ones_like(gx)], axis=-1).reshape(-1, 4)
    tp = jnp.einsum('bij,pj->bpi', M, pts)
    grid = (tp * scale).reshape(B, D_OUT, oH, oW, 3)
    crd = 128.0 * (grid * 0.5 + 0.5)
    crd = crd.reshape(B, D_OUT, oH, oW // LANES, LANES, 3)
    crd = jnp.transpose(crd, (0, 2, 3, 5, 1, 4))
    return crd.reshape(B * oH * (oW // LANES), 3, D_OUT, LANES)


def kernel(x, y, rtvec, corner_pt):
    B, C, D, H, W = x.shape
    oH, oW = y.shape[2], y.shape[3]
    M = _set_matrix(rtvec)
    dmin, dmax = _raydist_range(M, corner_pt, SRC)
    scale = 2.0 / (dmin + dmax + 1e-6)
    crd = _coords(M, scale, oH, oW, B)
    vol_flat = x.reshape(-1)
    out = _projector(vol_flat, crd)
    return out.reshape(B, C, oH, oW)


# speedup vs baseline: 2.6672x; 2.6672x over previous
import functools

import jax
import jax.numpy as jnp
from jax import lax
from jax.experimental import pallas as pl
from jax.experimental.pallas import tpu as pltpu
from jax.experimental.pallas import tpu_sc as plsc

SRC = 4.0
D_OUT = 64
NC, NS, LANES = 2, 16, 16
NW = NC * NS
VOL = 128 * 128 * 128
N_PIX = 4 * 128 * 128
PIX_PER_W = N_PIX // NW
GROUPS = PIX_PER_W // LANES


def _set_matrix(rtvec):
    B = rtvec.shape[0]
    rx, ry, rz = rtvec[:, 0], rtvec[:, 1], rtvec[:, 2]
    tx, ty, tz = rtvec[:, 3], rtvec[:, 4], rtvec[:, 5]
    z = jnp.zeros(B, dtype=rtvec.dtype); o = jnp.ones(B, dtype=rtvec.dtype)
    cx, sx = jnp.cos(rx), jnp.sin(rx)
    cy, sy = jnp.cos(ry), jnp.sin(ry)
    cz, sz = jnp.cos(rz), jnp.sin(rz)
    Rx = jnp.stack([o, z, z, z, z, cx, -sx, z, z, sx, cx, z, z, z, z, o], axis=1).reshape(B, 4, 4)
    Ry = jnp.stack([cy, z, sy, z, z, o, z, z, -sy, z, cy, z, z, z, z, o], axis=1).reshape(B, 4, 4)
    Rz = jnp.stack([cz, -sz, z, z, sz, cz, z, z, z, z, o, z, z, z, z, o], axis=1).reshape(B, 4, 4)
    T = jnp.stack([o, z, z, tx, z, o, z, ty, z, z, o, tz, z, z, z, o], axis=1).reshape(B, 4, 4)
    rot = jnp.einsum('bij,bjk->bik', jnp.einsum('bij,bjk->bik', Rz, Ry), Rx)
    M = jnp.einsum('bij,bjk->bik', rot, T)
    return M[:, :3, :]


def _raydist_range(M, pt, src):
    pt = pt - M[:, :3, 3][:, None, :]
    invR = jnp.linalg.inv(M[:, :3, :3])
    inv_pt = jnp.einsum('bnc,bcd->bnd', pt, invR)
    inv_pt = inv_pt.at[:, :, 2].set(src - inv_pt[:, :, 2])
    flat = inv_pt.reshape(-1, 3)
    d = jnp.sqrt(flat[:, 0] ** 2 + flat[:, 1] ** 2 + flat[:, 2] ** 2)
    return jnp.min(d), jnp.max(d)


def _sc_body(vol_hbm, crd_hbm, out_hbm, crd_v, idx_buf, w_buf, val_buf, out_v, sem):
    cid = lax.axis_index("c")
    sid = lax.axis_index("s")
    wid = sid * NC + cid
    b = wid // 8
    sub = wid - b * 8
    vol_base = b * VOL

    def phase_a(g, p):
        row = b * 1024 + (sub * 16 + g // 8) * 8 + (g - (g // 8) * 8)
        pltpu.sync_copy(crd_hbm.at[row], crd_v.at[p])

        def step(d, _):
            vx = crd_v[p, 0, d]
            vy = crd_v[p, 1, d]
            vz = crd_v[p, 2, d]
            x0i = jnp.clip(lax.convert_element_type(vx, jnp.int32), 0, 127)
            y0i = jnp.clip(lax.convert_element_type(vy, jnp.int32), 0, 127)
            z0i = jnp.clip(lax.convert_element_type(vz, jnp.int32), 0, 127)
            x0f = lax.convert_element_type(x0i, jnp.float32)
            y0f = lax.convert_element_type(y0i, jnp.float32)
            z0f = lax.convert_element_type(z0i, jnp.float32)
            x1f = jnp.minimum(x0f + 1.0, 127.0)
            y1f = jnp.minimum(y0f + 1.0, 127.0)
            z1f = jnp.minimum(z0f + 1.0, 127.0)
            x1i = lax.convert_element_type(x1f, jnp.int32)
            y1i = lax.convert_element_type(y1f, jnp.int32)
            z1i = lax.convert_element_type(z1f, jnp.int32)
            wx0 = x1f - vx; wx1 = vx - x0f
            wy0 = y1f - vy; wy1 = vy - y0f
            wz0 = z1f - vz; wz1 = vz - z0f
            inb = ((vx >= 0.0) & (vx <= 128.0) & (vy >= 0.0) & (vy <= 128.0)
                   & (vz >= 0.0) & (vz <= 128.0))
            msk = jnp.where(inb, 1.0, 0.0)
            wz0 = wz0 * msk
            wz1 = wz1 * msk
            r00 = vol_base + z0i * 16384 + y0i * 128
            r01 = vol_base + z0i * 16384 + y1i * 128
            r10 = vol_base + z1i * 16384 + y0i * 128
            r11 = vol_base + z1i * 16384 + y1i * 128
            idx_buf[p, d, 0:16] = r00 + x0i
            idx_buf[p, d, 16:32] = r00 + x1i
            idx_buf[p, d, 32:48] = r01 + x0i
            idx_buf[p, d, 48:64] = r01 + x1i
            idx_buf[p, d, 64:80] = r10 + x0i
            idx_buf[p, d, 80:96] = r10 + x1i
            idx_buf[p, d, 96:112] = r11 + x0i
            idx_buf[p, d, 112:128] = r11 + x1i
            w_buf[p, d, 0:16] = wx0
            w_buf[p, d, 16:32] = wx1
            w_buf[p, d, 32:48] = wy0
            w_buf[p, d, 48:64] = wy1
            w_buf[p, d, 64:80] = wz0
            w_buf[p, d, 80:96] = wz1
            return 0

        lax.fori_loop(0, D_OUT, step, 0)

    def fire(p):
        def step(d, _):
            pltpu.async_copy(
                vol_hbm.at[idx_buf.at[p, d]], val_buf.at[p, d], sem.at[p])
            return 0
        lax.fori_loop(0, D_OUT, step, 0)

    def drain_combine(p):
        def dstep(d, _):
            pltpu.make_async_copy(
                vol_hbm.at[pl.ds(0, 128)], val_buf.at[p, d], sem.at[p]).wait()
            return 0
        lax.fori_loop(0, D_OUT, dstep, 0)

        def cstep(d, acc):
            wx0 = w_buf[p, d, 0:16]; wx1 = w_buf[p, d, 16:32]
            wy0 = w_buf[p, d, 32:48]; wy1 = w_buf[p, d, 48:64]
            wz0 = w_buf[p, d, 64:80]; wz1 = w_buf[p, d, 80:96]
            va = val_buf[p, d, 0:16]; vb = val_buf[p, d, 16:32]
            vc = val_buf[p, d, 32:48]; vd = val_buf[p, d, 48:64]
            ve = val_buf[p, d, 64:80]; vf = val_buf[p, d, 80:96]
            vg = val_buf[p, d, 96:112]; vh = val_buf[p, d, 112:128]
            bot = wy0 * (wx0 * va + wx1 * vb) + wy1 * (wx0 * vc + wx1 * vd)
            top = wy0 * (wx0 * ve + wx1 * vf) + wy1 * (wx0 * vg + wx1 * vh)
            return acc + (wz0 * bot + wz1 * top)

        return lax.fori_loop(0, D_OUT, cstep, jnp.zeros((LANES,), jnp.float32))

    phase_a(0, 0)
    fire(0)

    def group(g, _):
        p = g & 1

        @pl.when(g + 1 < GROUPS)
        def _():
            phase_a(g + 1, 1 - p)
            fire(1 - p)

        acc = drain_combine(p)
        out_v[pl.ds(g * 16, 16)] = acc * (1.0 / D_OUT)
        return 0

    lax.fori_loop(0, GROUPS, group, 0)
    pltpu.sync_copy(out_v, out_hbm.at[pl.ds(wid * PIX_PER_W, PIX_PER_W)])


_MESH = plsc.VectorSubcoreMesh(
    core_axis_name="c", subcore_axis_name="s", num_cores=NC, num_subcores=NS)

_projector = functools.partial(
    pl.kernel,
    out_type=jax.ShapeDtypeStruct((N_PIX,), jnp.float32),
    mesh=_MESH,
    scratch_types=[
        pltpu.VMEM((2, 3, D_OUT, LANES), jnp.float32),
        pltpu.VMEM((2, D_OUT, 128), jnp.int32),
        pltpu.VMEM((2, D_OUT, 96), jnp.float32),
        pltpu.VMEM((2, D_OUT, 128), jnp.float32),
        pltpu.VMEM((PIX_PER_W,), jnp.float32),
        pltpu.SemaphoreType.DMA((2,)),
    ],
)(_sc_body)


def _coords(M, scale, oH, oW, B):
    zs = jnp.linspace(-1.0, 1.0, D_OUT)
    ys = jnp.linspace(-1.0, 1.0, oH)
    xs = jnp.linspace(-1.0, 1.0, oW)
    gz, gy, gx = jnp.meshgrid(zs, ys, xs, indexing='ij')
    pts = jnp.stack([gx, gy, gz, jnp.ones_like(gx)], axis=-1).reshape(-1, 4)
    tp = jnp.einsum('bij,pj->bpi', M, pts)
    grid = (tp * scale).reshape(B, D_OUT, oH, oW, 3)
    crd = 128.0 * (grid * 0.5 + 0.5)
    crd = crd.reshape(B, D_OUT, oH, oW // LANES, LANES, 3)
    crd = jnp.transpose(crd, (0, 2, 3, 5, 1, 4))
    return crd.reshape(B * oH * (oW // LANES), 3, D_OUT, LANES)


def kernel(x, y, rtvec, corner_pt):
    B, C, D, H, W = x.shape
    oH, oW = y.shape[2], y.shape[3]
    M = _set_matrix(rtvec)
    dmin, dmax = _raydist_range(M, corner_pt, SRC)
    scale = 2.0 / (dmin + dmax + 1e-6)
    crd = _coords(M, scale, oH, oW, B)
    vol_flat = x.reshape(-1)
    out = _projector(vol_flat, crd)
    return out.reshape(B, C, oH, oW)


# speedup vs baseline: 2.7779x; 1.0415x over previous
import functools

import jax
import jax.numpy as jnp
from jax import lax
from jax.experimental import pallas as pl
from jax.experimental.pallas import tpu as pltpu
from jax.experimental.pallas import tpu_sc as plsc

SRC = 4.0
D_OUT = 64
NC, NS, LANES = 2, 16, 16
NW = NC * NS
VOL = 128 * 128 * 128
N_PIX = 4 * 128 * 128
PIX_PER_W = N_PIX // NW
GROUPS = PIX_PER_W // LANES


def _set_matrix(rtvec):
    B = rtvec.shape[0]
    rx, ry, rz = rtvec[:, 0], rtvec[:, 1], rtvec[:, 2]
    tx, ty, tz = rtvec[:, 3], rtvec[:, 4], rtvec[:, 5]
    z = jnp.zeros(B, dtype=rtvec.dtype); o = jnp.ones(B, dtype=rtvec.dtype)
    cx, sx = jnp.cos(rx), jnp.sin(rx)
    cy, sy = jnp.cos(ry), jnp.sin(ry)
    cz, sz = jnp.cos(rz), jnp.sin(rz)
    Rx = jnp.stack([o, z, z, z, z, cx, -sx, z, z, sx, cx, z, z, z, z, o], axis=1).reshape(B, 4, 4)
    Ry = jnp.stack([cy, z, sy, z, z, o, z, z, -sy, z, cy, z, z, z, z, o], axis=1).reshape(B, 4, 4)
    Rz = jnp.stack([cz, -sz, z, z, sz, cz, z, z, z, z, o, z, z, z, z, o], axis=1).reshape(B, 4, 4)
    T = jnp.stack([o, z, z, tx, z, o, z, ty, z, z, o, tz, z, z, z, o], axis=1).reshape(B, 4, 4)
    rot = jnp.einsum('bij,bjk->bik', jnp.einsum('bij,bjk->bik', Rz, Ry), Rx)
    M = jnp.einsum('bij,bjk->bik', rot, T)
    return M[:, :3, :]


def _raydist_range(M, pt, src):
    pt = pt - M[:, :3, 3][:, None, :]
    invR = jnp.linalg.inv(M[:, :3, :3])
    inv_pt = jnp.einsum('bnc,bcd->bnd', pt, invR)
    inv_pt = inv_pt.at[:, :, 2].set(src - inv_pt[:, :, 2])
    flat = inv_pt.reshape(-1, 3)
    d = jnp.sqrt(flat[:, 0] ** 2 + flat[:, 1] ** 2 + flat[:, 2] ** 2)
    return jnp.min(d), jnp.max(d)


def _sc_body(vol_hbm, crd_hbm, dum_hbm, out_hbm,
             crd0, crd1, idx0, idx1, w0, w1, val0, val1, out_v, sem, csem):
    cid = lax.axis_index("c")
    sid = lax.axis_index("s")
    wid = sid * NC + cid
    b = wid // 8
    sub = wid - b * 8
    vol_base = b * VOL

    def load_crd(g, crd_s, cs):
        row = b * 1024 + (sub * 16 + g // 8) * 8 + (g - (g // 8) * 8)
        pltpu.async_copy(crd_hbm.at[row], crd_s, cs)

    def phase_a(crd_s, idx_s, w_s, cs):
        pltpu.make_async_copy(crd_hbm.at[0], crd_s, cs).wait()

        def step(d, _):
            vx = crd_s[0, d]
            vy = crd_s[1, d]
            vz = crd_s[2, d]
            x0i = jnp.clip(lax.convert_element_type(vx, jnp.int32), 0, 127)
            y0i = jnp.clip(lax.convert_element_type(vy, jnp.int32), 0, 127)
            z0i = jnp.clip(lax.convert_element_type(vz, jnp.int32), 0, 127)
            x0f = lax.convert_element_type(x0i, jnp.float32)
            y0f = lax.convert_element_type(y0i, jnp.float32)
            z0f = lax.convert_element_type(z0i, jnp.float32)
            x1f = jnp.minimum(x0f + 1.0, 127.0)
            y1f = jnp.minimum(y0f + 1.0, 127.0)
            z1f = jnp.minimum(z0f + 1.0, 127.0)
            x1i = lax.convert_element_type(x1f, jnp.int32)
            y1i = lax.convert_element_type(y1f, jnp.int32)
            z1i = lax.convert_element_type(z1f, jnp.int32)
            wx0 = x1f - vx; wx1 = vx - x0f
            wy0 = y1f - vy; wy1 = vy - y0f
            wz0 = z1f - vz; wz1 = vz - z0f
            inb = ((vx >= 0.0) & (vx <= 128.0) & (vy >= 0.0) & (vy <= 128.0)
                   & (vz >= 0.0) & (vz <= 128.0))
            msk = jnp.where(inb, 1.0, 0.0)
            wz0 = wz0 * msk
            wz1 = wz1 * msk
            r00 = vol_base + z0i * 16384 + y0i * 128
            r01 = vol_base + z0i * 16384 + y1i * 128
            r10 = vol_base + z1i * 16384 + y0i * 128
            r11 = vol_base + z1i * 16384 + y1i * 128
            o = d * 128
            idx_s[pl.ds(o + 0, 16)] = r00 + x0i
            idx_s[pl.ds(o + 16, 16)] = r00 + x1i
            idx_s[pl.ds(o + 32, 16)] = r01 + x0i
            idx_s[pl.ds(o + 48, 16)] = r01 + x1i
            idx_s[pl.ds(o + 64, 16)] = r10 + x0i
            idx_s[pl.ds(o + 80, 16)] = r10 + x1i
            idx_s[pl.ds(o + 96, 16)] = r11 + x0i
            idx_s[pl.ds(o + 112, 16)] = r11 + x1i
            w_s[d, 0:16] = wx0
            w_s[d, 16:32] = wx1
            w_s[d, 32:48] = wy0
            w_s[d, 48:64] = wy1
            w_s[d, 64:80] = wz0
            w_s[d, 80:96] = wz1
            return 0

        lax.fori_loop(0, D_OUT, step, 0)

    def fire(idx_s, val_s, s):
        pltpu.async_copy(vol_hbm.at[idx_s], val_s, s)

    def drain_combine(w_s, val_s, s):
        pltpu.make_async_copy(dum_hbm, val_s, s).wait()

        def cstep(d, acc):
            wx0 = w_s[d, 0:16]; wx1 = w_s[d, 16:32]
            wy0 = w_s[d, 32:48]; wy1 = w_s[d, 48:64]
            wz0 = w_s[d, 64:80]; wz1 = w_s[d, 80:96]
            o = d * 128
            va = val_s[pl.ds(o + 0, 16)]; vb = val_s[pl.ds(o + 16, 16)]
            vc = val_s[pl.ds(o + 32, 16)]; vd = val_s[pl.ds(o + 48, 16)]
            ve = val_s[pl.ds(o + 64, 16)]; vf = val_s[pl.ds(o + 80, 16)]
            vg = val_s[pl.ds(o + 96, 16)]; vh = val_s[pl.ds(o + 112, 16)]
            bot = wy0 * (wx0 * va + wx1 * vb) + wy1 * (wx0 * vc + wx1 * vd)
            top = wy0 * (wx0 * ve + wx1 * vf) + wy1 * (wx0 * vg + wx1 * vh)
            return acc + (wz0 * bot + wz1 * top)

        return lax.fori_loop(0, D_OUT, cstep, jnp.zeros((LANES,), jnp.float32))

    s0, s1 = sem.at[0], sem.at[1]
    c0, c1 = csem.at[0], csem.at[1]

    load_crd(0, crd0, c0)
    load_crd(1, crd1, c1)
    phase_a(crd0, idx0, w0, c0)
    fire(idx0, val0, s0)

    def pair(i, _):
        g = 2 * i

        @pl.when(g + 2 < GROUPS)
        def _():
            load_crd(g + 2, crd0, c0)

        phase_a(crd1, idx1, w1, c1)
        fire(idx1, val1, s1)
        acc = drain_combine(w0, val0, s0)
        out_v[pl.ds(g * 16, 16)] = acc * (1.0 / D_OUT)

        @pl.when(g + 3 < GROUPS)
        def _():
            load_crd(g + 3, crd1, c1)

        @pl.when(g + 2 < GROUPS)
        def _():
            phase_a(crd0, idx0, w0, c0)
            fire(idx0, val0, s0)

        acc = drain_combine(w1, val1, s1)
        out_v[pl.ds((g + 1) * 16, 16)] = acc * (1.0 / D_OUT)
        return 0

    lax.fori_loop(0, GROUPS // 2, pair, 0)
    pltpu.sync_copy(out_v, out_hbm.at[pl.ds(wid * PIX_PER_W, PIX_PER_W)])


_MESH = plsc.VectorSubcoreMesh(
    core_axis_name="c", subcore_axis_name="s", num_cores=NC, num_subcores=NS)

_projector = functools.partial(
    pl.kernel,
    out_type=jax.ShapeDtypeStruct((N_PIX,), jnp.float32),
    mesh=_MESH,
    scratch_types=[
        pltpu.VMEM((3, D_OUT, LANES), jnp.float32),
        pltpu.VMEM((3, D_OUT, LANES), jnp.float32),
        pltpu.VMEM((D_OUT * 128,), jnp.int32),
        pltpu.VMEM((D_OUT * 128,), jnp.int32),
        pltpu.VMEM((D_OUT, 96), jnp.float32),
        pltpu.VMEM((D_OUT, 96), jnp.float32),
        pltpu.VMEM((D_OUT * 128,), jnp.float32),
        pltpu.VMEM((D_OUT * 128,), jnp.float32),
        pltpu.VMEM((PIX_PER_W,), jnp.float32),
        pltpu.SemaphoreType.DMA((2,)),
        pltpu.SemaphoreType.DMA((2,)),
    ],
)(_sc_body)


def _coords(M, scale, oH, oW, B):
    zs = jnp.linspace(-1.0, 1.0, D_OUT)
    ys = jnp.linspace(-1.0, 1.0, oH)
    xs = jnp.linspace(-1.0, 1.0, oW)
    gz, gy, gx = jnp.meshgrid(zs, ys, xs, indexing='ij')
    pts = jnp.stack([gx, gy, gz, jnp.ones_like(gx)], axis=-1).reshape(-1, 4)
    tp = jnp.einsum('bij,pj->bpi', M, pts)
    grid = (tp * scale).reshape(B, D_OUT, oH, oW, 3)
    crd = 128.0 * (grid * 0.5 + 0.5)
    crd = crd.reshape(B, D_OUT, oH, oW // LANES, LANES, 3)
    crd = jnp.transpose(crd, (0, 2, 3, 5, 1, 4))
    return crd.reshape(B * oH * (oW // LANES), 3, D_OUT, LANES)


def kernel(x, y, rtvec, corner_pt):
    B, C, D, H, W = x.shape
    oH, oW = y.shape[2], y.shape[3]
    M = _set_matrix(rtvec)
    dmin, dmax = _raydist_range(M, corner_pt, SRC)
    scale = 2.0 / (dmin + dmax + 1e-6)
    crd = _coords(M, scale, oH, oW, B)
    vol_flat = x.reshape(-1)
    dum = jnp.zeros((D_OUT * 128,), jnp.float32)
    out = _projector(vol_flat, crd, dum)
    return out.reshape(B, C, oH, oW)
